# Initial kernel scaffold; baseline (speedup 1.0000x reference)
import jax
import jax.numpy as jnp
from jax.experimental import pallas as pl


def kernel(x, edge_index, W1l, b1l, W1r, W2l, b2l, W2r, Wc, bc):
    raise NotImplementedError("write your pallas kernel here")


# speedup vs baseline: 4.9811x; 4.9811x over previous
import dataclasses
import functools

import jax
import jax.numpy as jnp
from jax import lax
from jax.experimental import pallas as pl
from jax.experimental.pallas import tpu as pltpu
from jax.experimental.pallas import tpu_sc as plsc

N_NODES = 10000
N_PAD = 10240
N_EDGES = 320000
D_IN = 128
D_HID = 64

NC, NS = 2, 16
COLS_PER_TILE = D_HID // NS
TILE_WORDS = COLS_PER_TILE * N_PAD
E_PER_CORE = N_EDGES // NC
E_PER_TILE = N_EDGES // (NC * NS)
CHUNK = 4000
CNT_CHUNK = 2000
CBLK = 2048

_mesh = plsc.VectorSubcoreMesh(core_axis_name="c", subcore_axis_name="s")

_sc_params = pltpu.CompilerParams()
if "needs_layout_passes" in pltpu.CompilerParams.__dataclass_fields__:
  _sc_params = dataclasses.replace(_sc_params, needs_layout_passes=False)


def _sc_count(dst):
  @functools.partial(
      pl.kernel,
      out_type=jax.ShapeDtypeStruct((NC * NS, N_PAD), jnp.float32),
      mesh=_mesh,
      compiler_params=_sc_params,
      scratch_types=[
          pltpu.VMEM((CNT_CHUNK,), jnp.int32),
          pltpu.VMEM((N_PAD,), jnp.float32),
      ],
  )
  def k(dst_hbm, out_hbm, idx_v, cnt_v):
    c = lax.axis_index("c")
    s = lax.axis_index("s")
    wid = s * NC + c
    zeros = jnp.zeros((16,), jnp.float32)
    ones = jnp.ones((16,), jnp.float32)

    @pl.loop(0, N_PAD // 16)
    def _(i):
      cnt_v[pl.ds(i * 16, 16)] = zeros

    base = wid * E_PER_TILE

    @pl.loop(0, E_PER_TILE // CNT_CHUNK)
    def _(ch):
      pltpu.sync_copy(dst_hbm.at[pl.ds(base + ch * CNT_CHUNK, CNT_CHUNK)], idx_v)

      @pl.loop(0, CNT_CHUNK // 16)
      def _(i):
        d16 = idx_v[pl.ds(i * 16, 16)]
        plsc.addupdate_scatter(cnt_v, [d16], ones)

    pltpu.sync_copy(cnt_v, out_hbm.at[wid])

  return k(dst)


def _sc_segsum(table_flat, src, dst):
  @functools.partial(
      pl.kernel,
      out_type=jax.ShapeDtypeStruct((NC, D_HID * N_PAD), jnp.float32),
      mesh=_mesh,
      compiler_params=_sc_params,
      scratch_types=[
          pltpu.VMEM((TILE_WORDS,), jnp.float32),
          pltpu.VMEM((TILE_WORDS,), jnp.float32),
          pltpu.VMEM((CHUNK,), jnp.int32),
          pltpu.VMEM((CHUNK,), jnp.int32),
      ],
  )
  def k(tab_hbm, src_hbm, dst_hbm, out_hbm, tab_v, acc_v, src_v, dst_v):
    c = lax.axis_index("c")
    s = lax.axis_index("s")
    zeros = jnp.zeros((16,), jnp.float32)

    pltpu.sync_copy(tab_hbm.at[pl.ds(s * TILE_WORDS, TILE_WORDS)], tab_v)

    @pl.loop(0, TILE_WORDS // 16)
    def _(i):
      acc_v[pl.ds(i * 16, 16)] = zeros

    ebase = c * E_PER_CORE

    @pl.loop(0, E_PER_CORE // CHUNK)
    def _(ch):
      off = ebase + ch * CHUNK
      pltpu.sync_copy(src_hbm.at[pl.ds(off, CHUNK)], src_v)
      pltpu.sync_copy(dst_hbm.at[pl.ds(off, CHUNK)], dst_v)

      @pl.loop(0, CHUNK // 16)
      def _(i):
        s16 = src_v[pl.ds(i * 16, 16)]
        d16 = dst_v[pl.ds(i * 16, 16)]
        for j in range(COLS_PER_TILE):
          v = plsc.load_gather(tab_v, [s16 + (j * N_PAD)])
          plsc.addupdate_scatter(acc_v, [d16 + (j * N_PAD)], v)

    pltpu.sync_copy(acc_v, out_hbm.at[c, pl.ds(s * TILE_WORDS, TILE_WORDS)])

  return k(table_flat, src, dst)


_DN = (((1,), (0,)), ((), ()))
_PREC = lax.Precision.HIGHEST


def _tc_proj(WcatT, xT):
  def body(w_ref, x_ref, y_ref, r_ref):
    res = lax.dot_general(w_ref[...], x_ref[...], _DN, precision=_PREC,
                          preferred_element_type=jnp.float32)
    y_ref[...] = res[:D_HID]
    r_ref[...] = res[D_HID:]

  return pl.pallas_call(
      body,
      grid=(N_PAD // CBLK,),
      in_specs=[
          pl.BlockSpec((D_IN, D_IN), lambda j: (0, 0)),
          pl.BlockSpec((D_IN, CBLK), lambda j: (0, j)),
      ],
      out_specs=[
          pl.BlockSpec((D_HID, CBLK), lambda j: (0, j)),
          pl.BlockSpec((D_HID, CBLK), lambda j: (0, j)),
      ],
      out_shape=[
          jax.ShapeDtypeStruct((D_HID, N_PAD), jnp.float32),
          jax.ShapeDtypeStruct((D_HID, N_PAD), jnp.float32),
      ],
  )(WcatT, xT)


def _tc_mid(p, cntp, rT, b, W2catT):
  def body(p_ref, c_ref, r_ref, b_ref, w_ref, y_ref, rr_ref):
    agg = p_ref[0] + p_ref[1]
    cnt = jnp.sum(c_ref[...], axis=0)
    inv = 1.0 / jnp.maximum(cnt, 1.0)
    h = jnp.maximum(agg * inv[None, :] + b_ref[...] + r_ref[...], 0.0)
    res = lax.dot_general(w_ref[...], h, _DN, precision=_PREC,
                          preferred_element_type=jnp.float32)
    y_ref[...] = res[:D_HID]
    rr_ref[...] = res[D_HID:]

  return pl.pallas_call(
      body,
      grid=(N_PAD // CBLK,),
      in_specs=[
          pl.BlockSpec((NC, D_HID, CBLK), lambda j: (0, 0, j)),
          pl.BlockSpec((NC * NS, CBLK), lambda j: (0, j)),
          pl.BlockSpec((D_HID, CBLK), lambda j: (0, j)),
          pl.BlockSpec((D_HID, 1), lambda j: (0, 0)),
          pl.BlockSpec((D_IN, D_HID), lambda j: (0, 0)),
      ],
      out_specs=[
          pl.BlockSpec((D_HID, CBLK), lambda j: (0, j)),
          pl.BlockSpec((D_HID, CBLK), lambda j: (0, j)),
      ],
      out_shape=[
          jax.ShapeDtypeStruct((D_HID, N_PAD), jnp.float32),
          jax.ShapeDtypeStruct((D_HID, N_PAD), jnp.float32),
      ],
  )(p, cntp, rT, b, W2catT)


def _tc_final(p, cntp, rT, b, Wc, bc):
  def body(p_ref, c_ref, r_ref, b_ref, w_ref, bc_ref, o_ref):
    agg = p_ref[0] + p_ref[1]
    cnt = jnp.sum(c_ref[...], axis=0)
    inv = 1.0 / jnp.maximum(cnt, 1.0)
    h = jnp.maximum(agg * inv[None, :] + b_ref[...] + r_ref[...], 0.0)
    o_ref[...] = jnp.sum(h * w_ref[...], axis=0, keepdims=True) + bc_ref[0, 0]

  return pl.pallas_call(
      body,
      grid=(N_PAD // CBLK,),
      in_specs=[
          pl.BlockSpec((NC, D_HID, CBLK), lambda j: (0, 0, j)),
          pl.BlockSpec((NC * NS, CBLK), lambda j: (0, j)),
          pl.BlockSpec((D_HID, CBLK), lambda j: (0, j)),
          pl.BlockSpec((D_HID, 1), lambda j: (0, 0)),
          pl.BlockSpec((D_HID, 1), lambda j: (0, 0)),
          pl.BlockSpec((1, 1), lambda j: (0, 0)),
      ],
      out_specs=pl.BlockSpec((1, CBLK), lambda j: (0, j)),
      out_shape=jax.ShapeDtypeStruct((1, N_PAD), jnp.float32),
  )(p, cntp, rT, b, Wc, bc)


def kernel(x, edge_index, W1l, b1l, W1r, W2l, b2l, W2r, Wc, bc):
  src = edge_index[0].astype(jnp.int32)
  dst = edge_index[1].astype(jnp.int32)

  xT = jnp.pad(x.T, ((0, 0), (0, N_PAD - N_NODES)))
  W1catT = jnp.concatenate([W1l, W1r], axis=1).T
  W2catT = jnp.concatenate([W2l, W2r], axis=1).T

  cntp = _sc_count(dst)

  y1T, r1T = _tc_proj(W1catT, xT)
  p1 = _sc_segsum(y1T.reshape(-1), src, dst)
  p1 = p1.reshape(NC, D_HID, N_PAD)

  y2T, r2T = _tc_mid(p1, cntp, r1T, b1l.reshape(D_HID, 1), W2catT)
  p2 = _sc_segsum(y2T.reshape(-1), src, dst)
  p2 = p2.reshape(NC, D_HID, N_PAD)

  logits = _tc_final(p2, cntp, r2T, b2l.reshape(D_HID, 1), Wc,
                     bc.reshape(1, 1))
  return logits[0, :N_NODES]
